# baseline (device time: 14653 ns/iter reference)
import jax
import jax.numpy as jnp
from jax import lax
from jax.experimental import pallas as pl
from jax.experimental.pallas import tpu as pltpu

N_DEV = 4
C = 4


def kernel(t, W):
    m, k = t.shape
    _, n = W.shape
    mc = m // C

    def body(t_ref, w_ref, out_ref, comm_ref, send_sems, recv_sems):
        my = lax.axis_index("i")
        p1 = my ^ 1
        p2 = my ^ 2

        barrier_sem = pltpu.get_barrier_semaphore()
        for nbr in (p1, p2):
            pl.semaphore_signal(
                barrier_sem, inc=1,
                device_id=(nbr,), device_id_type=pl.DeviceIdType.MESH,
            )

        y = jnp.dot(
            t_ref[...].astype(jnp.bfloat16),
            w_ref[...].astype(jnp.bfloat16),
            preferred_element_type=jnp.float32,
        )
        comm_ref[0] = y.astype(jnp.bfloat16).reshape(C, mc, n)

        pl.semaphore_wait(barrier_sem, 2)

        r1 = [
            pltpu.make_async_remote_copy(
                src_ref=comm_ref.at[0, c],
                dst_ref=comm_ref.at[1, c],
                send_sem=send_sems.at[0, c],
                recv_sem=recv_sems.at[0, c],
                device_id=(p1,),
                device_id_type=pl.DeviceIdType.MESH,
            )
            for c in range(C)
        ]
        r2 = [
            pltpu.make_async_remote_copy(
                src_ref=comm_ref.at[2, c],
                dst_ref=comm_ref.at[3, c],
                send_sem=send_sems.at[1, c],
                recv_sem=recv_sems.at[1, c],
                device_id=(p2,),
                device_id_type=pl.DeviceIdType.MESH,
            )
            for c in range(C)
        ]
        for c in range(C):
            r1[c].start()
        for c in range(C):
            r1[c].wait_recv()
            comm_ref[2, c] = comm_ref[0, c] + comm_ref[1, c]
            r2[c].start()
        for c in range(C):
            r2[c].wait_recv()
            out_ref[c * mc:(c + 1) * mc, :] = comm_ref[2, c] + comm_ref[3, c]
        for c in range(C):
            r1[c].wait_send()
            r2[c].wait_send()

    return pl.pallas_call(
        body,
        out_shape=jax.ShapeDtypeStruct((m, n), jnp.bfloat16),
        in_specs=[
            pl.BlockSpec(memory_space=pltpu.VMEM),
            pl.BlockSpec(memory_space=pltpu.VMEM),
        ],
        out_specs=pl.BlockSpec(memory_space=pltpu.VMEM),
        scratch_shapes=[
            pltpu.VMEM((4, C, mc, n), jnp.bfloat16),
            pltpu.SemaphoreType.DMA((2, C)),
            pltpu.SemaphoreType.DMA((2, C)),
        ],
        compiler_params=pltpu.CompilerParams(collective_id=0),
    )(t, W)


# device time: 11061 ns/iter; 1.3247x vs baseline; 1.3247x over previous
import jax
import jax.numpy as jnp
from jax import lax
from jax.experimental import pallas as pl
from jax.experimental.pallas import tpu as pltpu

N_DEV = 4
C = 4


def kernel(t, W):
    m, k = t.shape
    _, n = W.shape
    mh = m // 2
    mc = mh // C

    def body(t_ref, w_ref, out_ref, comm_ref, send_sems, recv_sems):
        my = lax.axis_index("i")
        p1 = my ^ 1
        p3 = my ^ 3
        partners = {0: (p1, p3), 1: (p3, p1)}

        barrier_sem = pltpu.get_barrier_semaphore()
        for nbr in (p1, p3):
            pl.semaphore_signal(
                barrier_sem, inc=1,
                device_id=(nbr,), device_id_type=pl.DeviceIdType.MESH,
            )

        y = jnp.dot(
            t_ref[...].astype(jnp.bfloat16),
            w_ref[...].astype(jnp.bfloat16),
            preferred_element_type=jnp.float32,
        )
        pl.semaphore_wait(barrier_sem, 2)

        def row0(h, c):
            return h * mh + c * mc

        def mk(round_, h, c):
            return pltpu.make_async_remote_copy(
                src_ref=comm_ref.at[2 * round_, h, c],
                dst_ref=comm_ref.at[2 * round_ + 1, h, c],
                send_sem=send_sems.at[round_, h, c],
                recv_sem=recv_sems.at[round_, h, c],
                device_id=(partners[h][round_],),
                device_id_type=pl.DeviceIdType.MESH,
            )

        r1 = [[mk(0, h, c) for c in range(C)] for h in range(2)]
        r2 = [[mk(1, h, c) for c in range(C)] for h in range(2)]

        yb = y.astype(jnp.bfloat16)
        for c in range(C):
            for h in (0, 1):
                a = row0(h, c)
                comm_ref[0, h, c] = yb[a:a + mc]
                r1[h][c].start()
        for c in range(C):
            for h in (0, 1):
                r1[h][c].wait_recv()
                comm_ref[2, h, c] = comm_ref[0, h, c] + comm_ref[1, h, c]
                r2[h][c].start()
        for c in range(C):
            for h in (0, 1):
                r2[h][c].wait_recv()
                a = row0(h, c)
                out_ref[a:a + mc, :] = (
                    comm_ref[2, h, c].astype(jnp.float32)
                    + comm_ref[3, h, c].astype(jnp.float32)
                )
        for c in range(C):
            for h in (0, 1):
                r1[h][c].wait_send()
                r2[h][c].wait_send()

    return pl.pallas_call(
        body,
        out_shape=jax.ShapeDtypeStruct((m, n), jnp.float32),
        in_specs=[
            pl.BlockSpec(memory_space=pltpu.VMEM),
            pl.BlockSpec(memory_space=pltpu.VMEM),
        ],
        out_specs=pl.BlockSpec(memory_space=pltpu.VMEM),
        scratch_shapes=[
            pltpu.VMEM((4, 2, C, mc, n), jnp.bfloat16),
            pltpu.SemaphoreType.DMA((2, 2, C)),
            pltpu.SemaphoreType.DMA((2, 2, C)),
        ],
        compiler_params=pltpu.CompilerParams(collective_id=0),
    )(t, W)
